# Initial kernel scaffold; baseline (speedup 1.0000x reference)
import jax
import jax.numpy as jnp
from jax.experimental import pallas as pl


def kernel(deep, medium, shallow, bboxes, labels):
    raise NotImplementedError("write your pallas kernel here")


# speedup vs baseline: 5.6053x; 5.6053x over previous
import functools

import jax
import jax.numpy as jnp
from jax import lax
from jax.experimental import pallas as pl
from jax.experimental.pallas import tpu as pltpu

_NUM_CLASSES = 80
_ATTRS = _NUM_CLASSES + 5
_INPUT_SIZE = 416.0
_ANCHORS = ((10., 13.), (16., 30.), (33., 23.), (30., 61.), (62., 45.),
            (59., 119.), (116., 90.), (156., 198.), (373., 326.))
_MASK = ((6, 7, 8), (3, 4, 5), (0, 1, 2))
_BALANCE = (0.4, 1.0, 4.0)
_BOX_RATIO = 0.05
_OBJ_RATIO = 5.0
_CLS_RATIO = _NUM_CLASSES / 80.0
_IGNORE_THR = 0.5
_EPS = 1e-7
_M = 32
_A = 3
_KSTEPS = _ATTRS // 5


def _bce(p, t):
    p = jnp.clip(p, _EPS, 1.0 - _EPS)
    return -(t * jnp.log(p) + (1.0 - t) * jnp.log(1.0 - p))


def _layer_body(lt, H, W, bx_ref, by_ref, bw_ref, bh_ref, lab_ref,
                x0_ref, x1_ref, x2_ref, out_ref, labf_ref):
    HW = H * W
    B = bx_ref.shape[0]
    k = pl.program_id(0)
    xrefs = (x0_ref, x1_ref, x2_ref)
    stride = _INPUT_SIZE / H
    sa = tuple((aw / stride, ah / stride) for aw, ah in _ANCHORS)
    mids = _MASK[lt]
    Hf, Wf = float(H), float(W)

    @pl.when(k == 0)
    def _heavy():
        bxv = bx_ref[...]
        byv = by_ref[...]
        bwv = bw_ref[...]
        bhv = bh_ref[...]
        labv = lab_ref[...]
        gx = bxv * Wf
        gy = byv * Hf
        gw = jnp.maximum(bwv * Wf, 1e-6)
        gh = jnp.maximum(bhv * Hf, 1e-6)
        valid = (bwv > 1e-6) & (bhv > 1e-6)

        def anch_iou(aw, ah):
            inter = jnp.minimum(gw, aw) * jnp.minimum(gh, ah)
            union = gw * gh + aw * ah - inter
            return inter / jnp.maximum(union, 1e-9)

        b_iou = anch_iou(*sa[0])
        b_idx = jnp.zeros(gx.shape, jnp.int32)
        b_aw = jnp.full(gx.shape, sa[0][0], jnp.float32)
        b_ah = jnp.full(gx.shape, sa[0][1], jnp.float32)
        for i in range(1, 9):
            iou_i = anch_iou(*sa[i])
            upd = iou_i > b_iou
            b_iou = jnp.where(upd, iou_i, b_iou)
            b_idx = jnp.where(upd, i, b_idx)
            b_aw = jnp.where(upd, sa[i][0], b_aw)
            b_ah = jnp.where(upd, sa[i][1], b_ah)

        in_layer = ((b_idx == mids[0]) | (b_idx == mids[1]) | (b_idx == mids[2]))
        a_sel = jnp.where(b_idx == mids[0], 0,
                          jnp.where(b_idx == mids[1], 1, 2)).astype(jnp.int32)
        gi = jnp.clip(jnp.floor(gx).astype(jnp.int32), 0, W - 1)
        gj = jnp.clip(jnp.floor(gy).astype(jnp.int32), 0, H - 1)
        ok = valid & in_layer
        tx = gx - gi.astype(jnp.float32)
        ty = gy - gj.astype(jnp.float32)
        tw = jnp.log(gw / b_aw)
        th = jnp.log(gh / b_ah)
        scl = gw * gh / (Hf * Wf)
        g1x = gx - gw / 2
        g2x = gx + gw / 2
        g1y = gy - gh / 2
        g2y = gy + gh / 2
        area_g = (g2x - g1x) * (g2y - g1y)

        q = lax.broadcasted_iota(jnp.int32, (B, HW), 1)
        jj = q // W
        ii = q - jj * W
        iif = ii.astype(jnp.float32)
        jjf = jj.astype(jnp.float32)

        s1 = 0.0
        s2 = 0.0
        s3 = 0.0
        s4 = 0.0
        npos = 0.0
        for a in range(_A):
            X = xrefs[a]
            px = jax.nn.sigmoid(X[:, 0, 0, :])
            py = jax.nn.sigmoid(X[:, 0, 1, :])
            pw = X[:, 0, 2, :]
            ph = X[:, 0, 3, :]
            pco = jax.nn.sigmoid(X[:, 0, 4, :])
            law, lah = sa[mids[a]]
            bxg = px + iif
            byg = py + jjf
            bwg = jnp.exp(pw) * law
            bhg = jnp.exp(ph) * lah
            p1x = bxg - bwg / 2
            p2x = bxg + bwg / 2
            p1y = byg - bhg / 2
            p2y = byg + bhg / 2
            area_p = (p2x - p1x) * (p2y - p1y)
            okla = ok & (a_sel == a)

            miou = jnp.zeros((B, HW), jnp.float32)
            is_pos = jnp.zeros((B, HW), jnp.bool_)
            txg = jnp.zeros((B, HW), jnp.float32)
            tyg = jnp.zeros((B, HW), jnp.float32)
            twg = jnp.zeros((B, HW), jnp.float32)
            thg = jnp.zeros((B, HW), jnp.float32)
            sclg = jnp.zeros((B, HW), jnp.float32)
            labg = jnp.full((B, HW), -1, jnp.int32)
            for m in range(_M):
                sl = lambda arr: arr[:, m:m + 1]
                iw = jnp.maximum(jnp.minimum(p2x, sl(g2x)) - jnp.maximum(p1x, sl(g1x)), 0.0)
                ih = jnp.maximum(jnp.minimum(p2y, sl(g2y)) - jnp.maximum(p1y, sl(g1y)), 0.0)
                inter = iw * ih
                iou = inter / jnp.maximum(area_p + sl(area_g) - inter, 1e-9)
                iou = jnp.where(sl(valid), iou, 0.0)
                miou = jnp.maximum(miou, iou)
                mt = sl(okla) & (jj == sl(gj)) & (ii == sl(gi))
                is_pos = is_pos | mt
                txg = jnp.where(mt, sl(tx), txg)
                tyg = jnp.where(mt, sl(ty), tyg)
                twg = jnp.where(mt, sl(tw), twg)
                thg = jnp.where(mt, sl(th), thg)
                sclg = jnp.where(mt, sl(scl), sclg)
                labg = jnp.where(mt, sl(labv), labg)

            pos = is_pos.astype(jnp.float32)
            bsc = 2.0 - sclg
            s1 = s1 + jnp.sum(pos * bsc * (_bce(px, txg) + _bce(py, tyg)))
            s2 = s2 + jnp.sum(pos * bsc * 0.5 * ((pw - twg) ** 2 + (ph - thg) ** 2))
            s3 = s3 + jnp.sum(pos * _bce(pco, 1.0))
            noobj = (1.0 - pos) * (miou <= _IGNORE_THR).astype(jnp.float32)
            s4 = s4 + jnp.sum(noobj * _bce(pco, 0.0))
            npos = npos + jnp.sum(pos)
            labf_ref[a] = labg

        out_ref[0] = s1
        out_ref[1] = s2
        out_ref[2] = s3
        out_ref[3] = s4
        out_ref[4] = 0.0
        out_ref[5] = npos

    @pl.when(k != 0)
    def _cls():
        acc = 0.0
        for a in range(_A):
            X = xrefs[a]
            labg = labf_ref[a]
            posf = (labg >= 0).astype(jnp.float32)
            for c in range(5):
                cid = 5 * k - 5 + c
                p = jax.nn.sigmoid(X[:, 0, c, :])
                t = (labg == cid).astype(jnp.float32)
                acc = acc + jnp.sum(posf * _bce(p, t))
        out_ref[4] = out_ref[4] + acc


def _layer_sums(pred, bx, by, bw, bh, labels, lt):
    B, C, H, W = pred.shape
    HW = H * W
    x = pred.reshape(B, C // 5, 5, HW)
    vec_spec = pl.BlockSpec((B, _M), lambda k: (0, 0))

    def xspec(a):
        return pl.BlockSpec((B, 1, 5, HW), lambda k, a=a: (0, _KSTEPS * a + k, 0, 0))

    return pl.pallas_call(
        functools.partial(_layer_body, lt, H, W),
        grid=(_KSTEPS,),
        in_specs=[vec_spec, vec_spec, vec_spec, vec_spec, vec_spec,
                  xspec(0), xspec(1), xspec(2)],
        out_specs=pl.BlockSpec(memory_space=pltpu.SMEM),
        out_shape=jax.ShapeDtypeStruct((8,), jnp.float32),
        scratch_shapes=[pltpu.VMEM((_A, B, HW), jnp.int32)],
    )(bx, by, bw, bh, labels, x, x, x)


def kernel(deep, medium, shallow, bboxes, labels):
    bx = bboxes[..., 0]
    by = bboxes[..., 1]
    bw = bboxes[..., 2]
    bh = bboxes[..., 3]
    labels = labels.astype(jnp.int32)
    total = jnp.float32(0.0)
    for pred, lt in ((deep, 0), (medium, 1), (shallow, 2)):
        s = _layer_sums(pred, bx, by, bw, bh, labels, lt)
        n_pos = jnp.maximum(s[5], 1.0)
        l = (_BOX_RATIO * (s[0] + s[1])
             + _OBJ_RATIO * _BALANCE[lt] * (s[2] + s[3])
             + _CLS_RATIO * s[4]) / n_pos
        total = total + l
    return total
